# baseline (speedup 1.0000x reference)
import functools

import jax
import jax.numpy as jnp
from jax import lax
from jax.experimental import pallas as pl
from jax.experimental.pallas import tpu as pltpu
from jax.experimental.pallas import tpu_sc as plsc

_B = 16384
_D = 32
_NC = 2
_NS = 16
_NW = _NC * _NS
_V = 1000000
_BPW = _B // _NW
_CH = 128
_NCH = _BPW // _CH

_mesh = plsc.VectorSubcoreMesh(
    core_axis_name="c", subcore_axis_name="s",
    num_cores=_NC, num_subcores=_NS)


@functools.partial(
    pl.kernel,
    out_type=(
        jax.ShapeDtypeStruct((_D, _B), jnp.float32),
        jax.ShapeDtypeStruct((_D, _B), jnp.float32),
    ),
    mesh=_mesh,
    compiler_params=pltpu.CompilerParams(use_tc_tiling_on_sc=False),
    scratch_types=[
        pltpu.VMEM((_NCH, _CH), jnp.int32),
        pltpu.VMEM((_NCH, _CH), jnp.int32),
        pltpu.VMEM((_D, _BPW), jnp.float32),
        pltpu.VMEM((_D, _BPW), jnp.float32),
        pltpu.SemaphoreType.DMA,
    ],
)
def _double_gather_t(sr_hbm, tg_hbm, w_sr_t, w_tg_t,
                     out_sr_t, out_tg_t,
                     idx_sr, idx_tg, cols_sr, cols_tg, sem):
  wid = lax.axis_index("s") * _NC + lax.axis_index("c")
  base = wid * _BPW

  pltpu.sync_copy(sr_hbm.at[wid], idx_sr)
  pltpu.sync_copy(tg_hbm.at[wid], idx_tg)

  def body(c, _):
    cps = []
    for j in range(_NCH):
      cps.append(pltpu.async_copy(
          w_sr_t.at[c].at[idx_sr.at[j]],
          cols_sr.at[c, pl.ds(j * _CH, _CH)], sem))
      cps.append(pltpu.async_copy(
          w_tg_t.at[c].at[idx_tg.at[j]],
          cols_tg.at[c, pl.ds(j * _CH, _CH)], sem))
    for cp in cps:
      cp.wait()
    return 0

  lax.fori_loop(0, _D, body, 0)

  pltpu.sync_copy(cols_sr, out_sr_t.at[:, pl.ds(base, _BPW)])
  pltpu.sync_copy(cols_tg, out_tg_t.at[:, pl.ds(base, _BPW)])


def kernel(sr_data, tg_data, W_sr, W_tg):
  sr3 = sr_data.reshape(_NW, _NCH, _CH)
  tg3 = tg_data.reshape(_NW, _NCH, _CH)
  out_sr_t, out_tg_t = _double_gather_t(sr3, tg3, W_sr.T, W_tg.T)
  return (out_sr_t.T, out_tg_t.T)
